# baseline (speedup 1.0000x reference)
import functools

import jax
import jax.numpy as jnp
from jax import lax
from jax.experimental import pallas as pl
from jax.experimental.pallas import tpu as pltpu
from jax.experimental.pallas import tpu_sc as plsc

MIN_COV = 1e-6

B, K, C, PS = 2, 1024, 32, 16
H, W = 512, 512
WH = 8
XW = 128
CGS = 8
NCS = C // CGS
NXW = W // XW
NYW = H // WH
NROUNDS = B * NCS * NXW * (NYW // 32)
KP = K
SEW = NYW + 16
NBUF = 4


def _sc_scatter(patch_logits, packed, se):
  mesh = plsc.VectorSubcoreMesh(core_axis_name="c", subcore_axis_name="s")

  @functools.partial(
      pl.kernel,
      out_type=(
          jax.ShapeDtypeStruct((B, C, H, W), jnp.float32),
          jax.ShapeDtypeStruct((B, H, W), jnp.float32),
      ),
      mesh=mesh,
      compiler_params=pltpu.CompilerParams(needs_layout_passes=False),
      scratch_types=[
          pltpu.VMEM((KP,), jnp.int32),
          pltpu.VMEM((2 * SEW,), jnp.int32),
          pltpu.VMEM((CGS * WH, XW), jnp.float32),
          pltpu.VMEM((WH, XW), jnp.float32),
          [pltpu.VMEM((CGS, PS, PS), jnp.float32)] * NBUF,
          [pltpu.SemaphoreType.DMA] * NBUF,
          pltpu.SemaphoreType.DMA,
      ],
  )
  def scatter_kernel(patch_hbm, packed_hbm, se_hbm, out_hbm, cnt_hbm,
                     pk_v, se_v, canvas, cntw, bufs, sems, fsem):
    cid = lax.axis_index("c")
    sid = lax.axis_index("s")
    wid = sid * 2 + cid

    zeros16 = jnp.zeros((16,), jnp.float32)
    ones16 = jnp.ones((16,), jnp.float32)
    iota16 = lax.iota(jnp.int32, 16)

    def _scalar_at(ref, flat_idx):
      return plsc.load_gather(ref, [jnp.full((16,), flat_idx, jnp.int32)])[0]

    def round_body(t, _):
      b = t // (NROUNDS // B)
      rr = lax.rem(t, NROUNDS // B)
      cg = rr // (NROUNDS // (B * NCS))
      xw = lax.rem(rr, NXW)
      yw = lax.rem(rr // NXW, NYW // 32) * 32 + wid
      y0 = yw * WH
      x0 = xw * XW

      @pl.when(lax.rem(t, NROUNDS // B) == 0)
      def _load_tables():
        pltpu.sync_copy(packed_hbm.at[pl.ds(b * KP, KP)], pk_v)
        pltpu.sync_copy(se_hbm.at[pl.ds(b * 2 * SEW, 2 * SEW)], se_v)

      s = _scalar_at(se_v, yw)
      e = _scalar_at(se_v, SEW + yw)

      def colok(p):
        cc = p & 511
        return (cc >= x0 - (PS - 1)) & (cc <= x0 + XW - 1)

      def fetch(i, buf, sem):
        p = _scalar_at(pk_v, i)

        @pl.when(colok(p))
        def _go():
          k = lax.shift_right_logical(p, 18)
          pltpu.async_copy(patch_hbm.at[b, k, pl.ds(cg * CGS, CGS)], buf, sem)

      def consume(i, buf, sem):
        p = _scalar_at(pk_v, i)

        @pl.when(colok(p))
        def _go():
          pltpu.make_async_copy(patch_hbm.at[0, 0, pl.ds(0, CGS)], buf,
                                sem).wait()
          r = lax.shift_right_logical(p, 9) & 511
          cc = p & 511
          xidx = (cc - x0) + iota16
          xloc = jnp.clip(xidx, 0, XW - 1)
          msk = (xidx >= 0) & (xidx < XW)
          for dy in range(PS):
            yl = r + dy - y0
            ok = (yl >= 0) & (yl < WH)

            @pl.when(ok)
            def _add():
              for ch in range(CGS):
                rowv = jnp.full((16,), ch * WH + yl, jnp.int32)
                plsc.addupdate_scatter(canvas, [rowv, xloc],
                                       buf[ch, dy, :], mask=msk)

              @pl.when(cg == 0)
              def _cnt():
                ylv = jnp.full((16,), yl, jnp.int32)
                plsc.addupdate_scatter(cntw, [ylv, xloc], ones16, mask=msk)

      for j in range(NBUF):
        @pl.when(s + j < e)
        def _prime(j=j):
          fetch(s + j, bufs[j], sems[j])

      def zrow(q, _):
        for j in range(XW // 16):
          canvas[q, pl.ds(j * 16, 16)] = zeros16
        return 0
      lax.fori_loop(0, CGS * WH, zrow, 0)

      for y in range(WH):
        for j in range(XW // 16):
          cntw[y, pl.ds(j * 16, 16)] = zeros16

      def pgroup(q, _):
        i = s + NBUF * q
        for j in range(NBUF):
          @pl.when(i + j < e)
          def _one(j=j):
            consume(i + j, bufs[j], sems[j])

            @pl.when(i + j + NBUF < e)
            def _refill():
              fetch(i + j + NBUF, bufs[j], sems[j])

        return 0

      lax.fori_loop(0, (e - s + NBUF - 1) // NBUF, pgroup, 0)

      for ch in range(CGS):
        pltpu.async_copy(
            canvas.at[pl.ds(ch * WH, WH)],
            out_hbm.at[b, cg * CGS + ch, pl.ds(y0, WH), pl.ds(x0, XW)], fsem)

      @pl.when(cg == 0)
      def _flush_cnt():
        pltpu.async_copy(cntw, cnt_hbm.at[b, pl.ds(y0, WH), pl.ds(x0, XW)],
                         fsem)

      for ch in range(CGS):
        pltpu.make_async_copy(
            canvas.at[pl.ds(ch * WH, WH)],
            out_hbm.at[b, cg * CGS + ch, pl.ds(y0, WH), pl.ds(x0, XW)],
            fsem).wait()

      @pl.when(cg == 0)
      def _drain_cnt():
        pltpu.make_async_copy(cntw,
                              cnt_hbm.at[b, pl.ds(y0, WH), pl.ds(x0, XW)],
                              fsem).wait()

      return 0

    lax.fori_loop(0, NROUNDS, round_body, 0)

  return scatter_kernel(patch_logits, packed, se)


NCGN = 4


def _norm_body(can_ref, cnt_ref, prev_ref, out_ref):
  cnt = cnt_ref[...]
  covered = cnt > MIN_COV
  safe = jnp.maximum(cnt, MIN_COV)
  out_ref[...] = jnp.where(covered[:, None],
                           can_ref[...] / safe[:, None],
                           prev_ref[...])


def _normalize(canvas, counts, prev_pred):
  cgn = C // NCGN
  grid = (B, NCGN)
  return pl.pallas_call(
      _norm_body,
      grid=grid,
      in_specs=[
          pl.BlockSpec((1, cgn, H, W), lambda b, g: (b, g, 0, 0)),
          pl.BlockSpec((1, H, W), lambda b, g: (b, 0, 0)),
          pl.BlockSpec((1, cgn, H, W), lambda b, g: (b, g, 0, 0)),
      ],
      out_specs=pl.BlockSpec((1, cgn, H, W), lambda b, g: (b, g, 0, 0)),
      out_shape=jax.ShapeDtypeStruct((B, C, H, W), jnp.float32),
  )(canvas, counts, prev_pred)


def kernel(patch_logits, coords, output_size, prev_pred):
  del output_size
  r = coords[:, :, 0].astype(jnp.int32)
  cc = coords[:, :, 1].astype(jnp.int32)
  order = jnp.argsort(r, axis=1).astype(jnp.int32)
  r_s = jnp.take_along_axis(r, order, axis=1)
  c_s = jnp.take_along_axis(cc, order, axis=1)
  packed = (order << 18) | (r_s << 9) | c_s

  rv = jnp.arange(NYW, dtype=jnp.int32) * WH
  starts = jax.vmap(lambda rs: jnp.searchsorted(rs, rv - (PS - 1)))(r_s)
  ends = jax.vmap(lambda rs: jnp.searchsorted(rs, rv + WH))(r_s)
  se = jnp.stack([starts, ends], axis=1).astype(jnp.int32)
  se = jnp.pad(se, ((0, 0), (0, 0), (0, 16))).reshape(-1)
  packed = packed.reshape(-1)

  canvas, counts = _sc_scatter(patch_logits, packed, se)
  return _normalize(canvas, counts, prev_pred)


# speedup vs baseline: 1.8782x; 1.8782x over previous
import functools

import jax
import jax.numpy as jnp
from jax import lax
from jax.experimental import pallas as pl
from jax.experimental.pallas import tpu as pltpu
from jax.experimental.pallas import tpu_sc as plsc

MIN_COV = 1e-6

B, K, C, PS = 2, 1024, 32, 16
H, W = 512, 512
WH = 16
CG = 4
NCG = C // CG
NYW = H // WH
NROUNDS = B * NCG
KP = K + 16
SEW = NYW + 16


def _sc_scatter(patch_logits, packed, se):
  mesh = plsc.VectorSubcoreMesh(core_axis_name="c", subcore_axis_name="s")

  @functools.partial(
      pl.kernel,
      out_type=(
          jax.ShapeDtypeStruct((B, C, H * W), jnp.float32),
          jax.ShapeDtypeStruct((B, H * W), jnp.float32),
      ),
      mesh=mesh,
      compiler_params=pltpu.CompilerParams(needs_layout_passes=False),
      scratch_types=[
          pltpu.VMEM((B * KP,), jnp.int32),
          pltpu.VMEM((B * 2 * SEW,), jnp.int32),
          pltpu.VMEM((CG * WH * W,), jnp.float32),
          pltpu.VMEM((WH * W,), jnp.float32),
          [pltpu.VMEM((CG, PS, PS), jnp.float32)] * 4,
          [pltpu.SemaphoreType.DMA] * 4,
          pltpu.SemaphoreType.DMA,
      ],
  )
  def scatter_kernel(patch_hbm, packed_hbm, se_hbm, out_hbm, cnt_hbm,
                     pk_v, se_v, canvas, cntw, bufs, sems, fsem):
    cid = lax.axis_index("c")
    sid = lax.axis_index("s")
    wid = sid * 2 + cid
    rbase = wid * WH

    pltpu.sync_copy(packed_hbm, pk_v)
    pltpu.sync_copy(se_hbm, se_v)

    zeros16 = jnp.zeros((16,), jnp.float32)
    ones16 = jnp.ones((16,), jnp.float32)
    iota16 = lax.iota(jnp.int32, 16)

    def _scalar_at(ref, flat_idx):
      return plsc.load_gather(ref, [jnp.full((16,), flat_idx, jnp.int32)])[0]

    def round_body(t, _):
      b = t // NCG
      cg = lax.rem(t, NCG)

      s = _scalar_at(se_v, (b * 2 + 0) * SEW + wid)
      e = _scalar_at(se_v, (b * 2 + 1) * SEW + wid)

      def fetch(i, buf, sem):
        p = _scalar_at(pk_v, b * KP + i)
        k = lax.shift_right_logical(p, 18)
        pltpu.async_copy(patch_hbm.at[b, k, pl.ds(cg * CG, CG)], buf, sem)

      def wait_buf(buf, sem):
        pltpu.make_async_copy(patch_hbm.at[0, 0, pl.ds(0, CG)], buf,
                              sem).wait()

      def scat(i, buf):
        p = _scalar_at(pk_v, b * KP + i)
        r = lax.shift_right_logical(p, 9) & 511
        cc = p & 511
        xidx = cc + iota16
        for dy in range(PS):
          yl = r + dy - rbase
          ok = (yl >= 0) & (yl < WH)

          @pl.when(ok)
          def _add():
            idx0 = yl * W + xidx
            for ch in range(CG):
              plsc.addupdate_scatter(canvas, [idx0 + ch * (WH * W)],
                                     buf[ch, dy, :])

            @pl.when(cg == 0)
            def _cnt():
              plsc.addupdate_scatter(cntw, [idx0], ones16)

      for j in range(4):
        @pl.when(s + j < e)
        def _prime(j=j):
          fetch(s + j, bufs[j], sems[j])

      def zrow(q, _):
        base = pl.multiple_of(q * 256, 256)
        for j in range(16):
          canvas[pl.ds(base + j * 16, 16)] = zeros16
        return 0
      lax.fori_loop(0, CG * WH * W // 256, zrow, 0)

      def zcnt(q, _):
        base = pl.multiple_of(q * 256, 256)
        for j in range(16):
          cntw[pl.ds(base + j * 16, 16)] = zeros16
        return 0
      lax.fori_loop(0, WH * W // 256, zcnt, 0)

      def pgroup(q, _):
        i = s + 4 * q
        for j in range(4):
          @pl.when(i + j < e)
          def _one(j=j):
            wait_buf(bufs[j], sems[j])
            scat(i + j, bufs[j])

            @pl.when(i + j + 4 < e)
            def _refill():
              fetch(i + j + 4, bufs[j], sems[j])

        return 0

      lax.fori_loop(0, (e - s + 3) // 4, pgroup, 0)

      for ch in range(CG):
        pltpu.async_copy(
            canvas.at[pl.ds(ch * WH * W, WH * W)],
            out_hbm.at[b, cg * CG + ch, pl.ds(rbase * W, WH * W)], fsem)

      @pl.when(cg == 0)
      def _flush_cnt():
        pltpu.async_copy(cntw, cnt_hbm.at[b, pl.ds(rbase * W, WH * W)], fsem)

      for ch in range(CG):
        pltpu.make_async_copy(
            canvas.at[pl.ds(ch * WH * W, WH * W)],
            out_hbm.at[b, cg * CG + ch, pl.ds(rbase * W, WH * W)],
            fsem).wait()

      @pl.when(cg == 0)
      def _drain_cnt():
        pltpu.make_async_copy(cntw, cnt_hbm.at[b, pl.ds(rbase * W, WH * W)],
                              fsem).wait()

      return 0

    lax.fori_loop(0, NROUNDS, round_body, 0)

  return scatter_kernel(patch_logits, packed, se)


def _norm_body(can_ref, cnt_ref, prev_ref, out_ref):
  cnt = cnt_ref[...]
  covered = cnt > MIN_COV
  safe = jnp.maximum(cnt, MIN_COV)
  out_ref[...] = jnp.where(covered[:, None],
                           can_ref[...] / safe[:, None],
                           prev_ref[...])


def _normalize(canvas, counts, prev_pred):
  grid = (B, NCG)
  return pl.pallas_call(
      _norm_body,
      grid=grid,
      in_specs=[
          pl.BlockSpec((1, CG, H, W), lambda b, g: (b, g, 0, 0)),
          pl.BlockSpec((1, H, W), lambda b, g: (b, 0, 0)),
          pl.BlockSpec((1, CG, H, W), lambda b, g: (b, g, 0, 0)),
      ],
      out_specs=pl.BlockSpec((1, CG, H, W), lambda b, g: (b, g, 0, 0)),
      out_shape=jax.ShapeDtypeStruct((B, C, H, W), jnp.float32),
  )(canvas, counts, prev_pred)


def kernel(patch_logits, coords, output_size, prev_pred):
  del output_size
  r = coords[:, :, 0].astype(jnp.int32)
  cc = coords[:, :, 1].astype(jnp.int32)
  order = jnp.argsort(r, axis=1).astype(jnp.int32)
  r_s = jnp.take_along_axis(r, order, axis=1)
  c_s = jnp.take_along_axis(cc, order, axis=1)
  packed = (order << 18) | (r_s << 9) | c_s

  rv = jnp.arange(NYW, dtype=jnp.int32) * WH
  starts = jax.vmap(lambda rs: jnp.searchsorted(rs, rv - (PS - 1)))(r_s)
  ends = jax.vmap(lambda rs: jnp.searchsorted(rs, rv + WH))(r_s)
  se = jnp.stack([starts, ends], axis=1).astype(jnp.int32)
  se = jnp.pad(se, ((0, 0), (0, 0), (0, 16))).reshape(-1)
  packed = jnp.pad(packed, ((0, 0), (0, 16))).reshape(-1)

  canvas, counts = _sc_scatter(patch_logits, packed, se)
  canvas = canvas.reshape(B, C, H, W)
  counts = counts.reshape(B, H, W)
  return _normalize(canvas, counts, prev_pred)
